# baseline (speedup 1.0000x reference)
import jax
import jax.numpy as jnp
from jax.experimental import pallas as pl
from jax.experimental.pallas import tpu as pltpu

_N = 50000
_C_IN = 128
_C_OUT = 128
_T = 8
_BLOCK = 1000


def _body(x_ref, t_ref, w_ref, o_ref):
    n = x_ref.shape[0]
    xb = jnp.concatenate(
        [x_ref[...].astype(jnp.bfloat16),
         jnp.ones((n, 8), jnp.bfloat16)], axis=1)
    tv = jnp.transpose(t_ref[...].reshape(1, n), (1, 0))
    yflat = jnp.dot(xb, w_ref[...], preferred_element_type=jnp.float32)
    ys = [yflat[:, t * _C_OUT:(t + 1) * _C_OUT] for t in range(_T)]
    m0 = (tv & 1) != 0
    m1 = (tv & 2) != 0
    m2 = (tv & 4) != 0
    a0 = jnp.where(m0, ys[1], ys[0])
    a1 = jnp.where(m0, ys[3], ys[2])
    a2 = jnp.where(m0, ys[5], ys[4])
    a3 = jnp.where(m0, ys[7], ys[6])
    c0 = jnp.where(m1, a1, a0)
    c1 = jnp.where(m1, a3, a2)
    o_ref[...] = jnp.where(m2, c1, c0)


def kernel(x, type_vec, W, b):
    n = x.shape[0]
    grid = n // _BLOCK
    t3 = type_vec.reshape(grid, 1, _BLOCK)
    w2 = jnp.concatenate(
        [W, b[:, None, :], jnp.zeros((_T, 7, _C_OUT), W.dtype)], axis=1)
    wb = w2.astype(jnp.bfloat16).transpose(1, 0, 2).reshape(
        _C_IN + 8, _T * _C_OUT)
    return pl.pallas_call(
        _body,
        grid=(grid,),
        in_specs=[
            pl.BlockSpec((_BLOCK, _C_IN), lambda i: (i, 0)),
            pl.BlockSpec((1, 1, _BLOCK), lambda i: (i, 0, 0)),
            pl.BlockSpec((_C_IN + 8, _T * _C_OUT), lambda i: (0, 0)),
        ],
        out_specs=pl.BlockSpec((_BLOCK, _C_OUT), lambda i: (i, 0)),
        out_shape=jax.ShapeDtypeStruct((n, _C_OUT), jnp.float32),
        compiler_params=pltpu.CompilerParams(
            dimension_semantics=("parallel",)),
    )(x, t3, wb)


# speedup vs baseline: 1.4394x; 1.4394x over previous
import jax
import jax.numpy as jnp
from jax.experimental import pallas as pl
from jax.experimental.pallas import tpu as pltpu

_N = 50000
_C_IN = 128
_C_OUT = 128
_T = 8
_BLOCK = 5000


def _body(x_ref, t_ref, w_ref, o_ref):
    n = x_ref.shape[0]
    xb = jnp.concatenate(
        [x_ref[...].astype(jnp.bfloat16),
         jnp.ones((n, 8), jnp.bfloat16)], axis=1)
    tv = jnp.transpose(t_ref[...].reshape(1, n), (1, 0))
    yflat = jnp.dot(xb, w_ref[...], preferred_element_type=jnp.float32)
    ys = [yflat[:, t * _C_OUT:(t + 1) * _C_OUT] for t in range(_T)]
    m0 = (tv & 1) != 0
    m1 = (tv & 2) != 0
    m2 = (tv & 4) != 0
    a0 = jnp.where(m0, ys[1], ys[0])
    a1 = jnp.where(m0, ys[3], ys[2])
    a2 = jnp.where(m0, ys[5], ys[4])
    a3 = jnp.where(m0, ys[7], ys[6])
    c0 = jnp.where(m1, a1, a0)
    c1 = jnp.where(m1, a3, a2)
    o_ref[...] = jnp.where(m2, c1, c0)


def kernel(x, type_vec, W, b):
    n = x.shape[0]
    grid = n // _BLOCK
    t3 = type_vec.reshape(grid, 1, _BLOCK)
    w2 = jnp.concatenate(
        [W, b[:, None, :], jnp.zeros((_T, 7, _C_OUT), W.dtype)], axis=1)
    wb = w2.astype(jnp.bfloat16).transpose(1, 0, 2).reshape(
        _C_IN + 8, _T * _C_OUT)
    return pl.pallas_call(
        _body,
        grid=(grid,),
        in_specs=[
            pl.BlockSpec((_BLOCK, _C_IN), lambda i: (i, 0)),
            pl.BlockSpec((1, 1, _BLOCK), lambda i: (i, 0, 0)),
            pl.BlockSpec((_C_IN + 8, _T * _C_OUT), lambda i: (0, 0)),
        ],
        out_specs=pl.BlockSpec((_BLOCK, _C_OUT), lambda i: (i, 0)),
        out_shape=jax.ShapeDtypeStruct((n, _C_OUT), jnp.float32),
        compiler_params=pltpu.CompilerParams(
            dimension_semantics=("parallel",)),
    )(x, t3, wb)
